# baseline (speedup 1.0000x reference)
import functools

import jax
import jax.numpy as jnp
from jax import lax
from jax.experimental import pallas as pl
from jax.experimental.pallas import tpu as pltpu
from jax.experimental.pallas import tpu_sc as plsc

_L = 16
_B = 128
_K = 4096


def _floor_i32(x):
    t = x.astype(jnp.int32)
    return jnp.where(x < t.astype(jnp.float32), t - 1, t)


def _feature_level_sc(ux, uy, g0f, t1, n, res0, res1, c0, c1):
    nworkers = 32
    per_w = n // nworkers
    steps = per_w // _B
    cout = 4 * c0 + c1
    npos = res0 * res0
    plane = npos
    pos_per_tile = npos // 16
    blocks = pos_per_tile // _K

    mesh = plsc.VectorSubcoreMesh(core_axis_name="c", subcore_axis_name="s")

    @functools.partial(
        pl.kernel,
        mesh=mesh,
        compiler_params=pltpu.CompilerParams(use_tc_tiling_on_sc=False,
                                             needs_layout_passes=False),
        out_type=jax.ShapeDtypeStruct((n, cout), jnp.float32),
        scratch_types=[
            pltpu.VMEM((_B,), jnp.float32),
            pltpu.VMEM((_B,), jnp.float32),
            pltpu.VMEM((8, _B), jnp.int32),
            pltpu.VMEM((4, _B), jnp.float32),
            pltpu.VMEM((_B, 8), jnp.float32),
            pltpu.VMEM((_B, 8), jnp.float32),
            pltpu.VMEM((_B, 8), jnp.float32),
            pltpu.VMEM((_B, 8), jnp.float32),
            pltpu.VMEM((_B, 16), jnp.float32),
            pltpu.VMEM((_B, 16), jnp.float32),
            pltpu.VMEM((_B, 16), jnp.float32),
            pltpu.VMEM((_B, 16), jnp.float32),
            pltpu.VMEM((_B, 48), jnp.float32),
            pltpu.VMEM((8, _K), jnp.float32),
            pltpu.VMEM((_K, 8), jnp.float32),
            pltpu.HBM((2, npos, 8), jnp.float32),
            pltpu.SemaphoreType.DMA,
        ],
    )
    def k(ux_hbm, uy_hbm, g0_hbm, t1_hbm, out_hbm,
          uxv, uyv, idx, wts, c00, c01, c10, c11, s00, s01, s10, s11,
          outv, tp, to, tbl, sem):
        sid = lax.axis_index("s")
        cc = lax.axis_index("c")
        wid = sid * 2 + cc

        def tblock(blk, _):
            pos0 = sid * pos_per_tile + blk * _K
            for c in range(8):
                pltpu.sync_copy(g0_hbm.at[pl.ds(c * plane + pos0, _K)],
                                tp.at[c])

            def shuffle(gg, _):
                lane = lax.iota(jnp.int32, _L)
                srows = lane & 7
                ssel = lane >> 3
                for u in range(8):
                    g = gg * 8 + u
                    v = plsc.load_gather(tp, [srows, ssel + 2 * g])
                    plsc.store_scatter(to, [ssel + 2 * g, srows], v)
                return 0

            lax.fori_loop(0, (_K * 8 // _L) // 8, shuffle, 0)
            pltpu.sync_copy(to, tbl.at[cc, pl.ds(pos0, _K)])
            return 0

        lax.fori_loop(0, blocks, tblock, 0)
        plsc.subcore_barrier()

        def step(st, _):
            base = wid * per_w + st * _B
            pltpu.sync_copy(ux_hbm.at[pl.ds(base, _B)], uxv)
            pltpu.sync_copy(uy_hbm.at[pl.ds(base, _B)], uyv)

            for g in range(_B // _L):
                sl = pl.ds(g * _L, _L)
                x = uxv[sl]
                y = uyv[sl]
                fx = _floor_i32(x * res0 - 0.5)
                fy = _floor_i32(y * res0 - 0.5)
                x0 = jnp.clip(fx, 0, res0 - 2)
                y0 = jnp.clip(fy, 0, res0 - 2)
                b00 = y0 * res0 + x0
                idx[0, sl] = b00
                idx[1, sl] = b00 + 1
                idx[2, sl] = b00 + res0
                idx[3, sl] = b00 + res0 + 1
                qx = x * res1 - 0.5
                qy = y * res1 - 0.5
                ix0 = _floor_i32(qx)
                iy0 = _floor_i32(qy)
                wx1 = qx - ix0.astype(jnp.float32)
                wy1 = qy - iy0.astype(jnp.float32)
                wx0 = 1.0 - wx1
                wy0 = 1.0 - wy1
                wx0 = jnp.where(ix0 >= 0, wx0, 0.0)
                wy0 = jnp.where(iy0 >= 0, wy0, 0.0)
                wx1 = jnp.where(ix0 + 1 <= res1 - 1, wx1, 0.0)
                wy1 = jnp.where(iy0 + 1 <= res1 - 1, wy1, 0.0)
                jx0 = jnp.maximum(ix0, 0)
                jy0 = jnp.maximum(iy0, 0)
                jx1 = jnp.minimum(ix0 + 1, res1 - 1)
                jy1 = jnp.minimum(iy0 + 1, res1 - 1)
                idx[4, sl] = jy0 * res1 + jx0
                idx[5, sl] = jy0 * res1 + jx1
                idx[6, sl] = jy1 * res1 + jx0
                idx[7, sl] = jy1 * res1 + jx1
                wts[0, sl] = wy0 * wx0
                wts[1, sl] = wy0 * wx1
                wts[2, sl] = wy1 * wx0
                wts[3, sl] = wy1 * wx1

            cps = [
                pltpu.async_copy(tbl.at[cc].at[idx.at[0]], c00, sem),
                pltpu.async_copy(tbl.at[cc].at[idx.at[1]], c01, sem),
                pltpu.async_copy(tbl.at[cc].at[idx.at[2]], c10, sem),
                pltpu.async_copy(tbl.at[cc].at[idx.at[3]], c11, sem),
                pltpu.async_copy(t1_hbm.at[idx.at[4]], s00, sem),
                pltpu.async_copy(t1_hbm.at[idx.at[5]], s01, sem),
                pltpu.async_copy(t1_hbm.at[idx.at[6]], s10, sem),
                pltpu.async_copy(t1_hbm.at[idx.at[7]], s11, sem),
            ]
            for cp in cps:
                cp.wait()

            def group(g, _):
                g16 = g * _L
                lane = lax.iota(jnp.int32, _L)
                rowsel = lane >> 3
                colsrc = lane & 7
                for p in range(_L // 2):
                    rows = rowsel + (g16 + 2 * p)
                    for kk, cbuf in enumerate((c00, c01, c10, c11)):
                        v = plsc.load_gather(cbuf, [rows, colsrc])
                        plsc.store_scatter(outv, [rows, colsrc + 8 * kk], v)
                wv = [wts[kk, pl.ds(g16, _L)] for kk in range(4)]
                for t in range(_L):
                    i = g16 + t
                    acc = (s00[i, :] * wv[0][t] + s01[i, :] * wv[1][t]
                           + s10[i, :] * wv[2][t] + s11[i, :] * wv[3][t])
                    outv[i, 32:48] = acc
                return 0

            lax.fori_loop(0, _B // _L, group, 0)
            pltpu.sync_copy(outv, out_hbm.at[pl.ds(base, _B)])
            return 0

        lax.fori_loop(0, steps, step, 0)

    return k(ux, uy, g0f, t1)


def kernel(uv, g0, g1):
    c0, res0 = g0.shape[1], g0.shape[2]
    c1, res1 = g1.shape[1], g1.shape[2]
    n = uv.shape[0]
    g0f = g0.reshape(c0 * res0 * res0)
    t1 = jnp.transpose(g1[0], (1, 2, 0)).reshape(res1 * res1, c1)
    ux = uv[:, 0] + 0.0
    uy = uv[:, 1] + 0.0
    return _feature_level_sc(ux, uy, g0f, t1, n, res0, res1, c0, c1)


# speedup vs baseline: 1.1443x; 1.1443x over previous
import functools

import jax
import jax.numpy as jnp
from jax import lax
from jax.experimental import pallas as pl
from jax.experimental.pallas import tpu as pltpu
from jax.experimental.pallas import tpu_sc as plsc

_L = 16
_B = 128
_K = 2048


def _floor_i32(x):
    t = x.astype(jnp.int32)
    return jnp.where(x < t.astype(jnp.float32), t - 1, t)


def _feature_level_sc(ux, uy, g0f, t1, n, res0, res1, c0, c1):
    nworkers = 32
    per_w = n // nworkers
    steps = per_w // _B
    cout = 4 * c0 + c1
    npos = res0 * res0
    plane = npos
    pos_per_tile = npos // 16
    blocks = pos_per_tile // _K

    mesh = plsc.VectorSubcoreMesh(core_axis_name="c", subcore_axis_name="s")

    @functools.partial(
        pl.kernel,
        mesh=mesh,
        compiler_params=pltpu.CompilerParams(use_tc_tiling_on_sc=False,
                                             needs_layout_passes=False),
        out_type=jax.ShapeDtypeStruct((n, cout), jnp.float32),
        scratch_types=[
            pltpu.VMEM((_B,), jnp.float32),
            pltpu.VMEM((_B,), jnp.float32),
            pltpu.VMEM((8, _B), jnp.int32),
            pltpu.VMEM((4, _B), jnp.float32),
            pltpu.VMEM((_B, 8), jnp.float32),
            pltpu.VMEM((_B, 8), jnp.float32),
            pltpu.VMEM((_B, 8), jnp.float32),
            pltpu.VMEM((_B, 8), jnp.float32),
            pltpu.VMEM((_B, 16), jnp.float32),
            pltpu.VMEM((_B, 16), jnp.float32),
            pltpu.VMEM((_B, 16), jnp.float32),
            pltpu.VMEM((_B, 16), jnp.float32),
            pltpu.VMEM((_B, 48), jnp.float32),
            pltpu.VMEM((8, _K), jnp.float32),
            pltpu.VMEM((8, _K), jnp.float32),
            pltpu.VMEM((_K, 8), jnp.float32),
            pltpu.VMEM((_K, 8), jnp.float32),
            pltpu.HBM((2, npos, 8), jnp.float32),
            pltpu.SemaphoreType.DMA,
            pltpu.SemaphoreType.DMA,
            pltpu.SemaphoreType.DMA,
            pltpu.SemaphoreType.DMA,
            pltpu.SemaphoreType.DMA,
        ],
    )
    def k(ux_hbm, uy_hbm, g0_hbm, t1_hbm, out_hbm,
          uxv, uyv, idx, wts, c00, c01, c10, c11, s00, s01, s10, s11,
          outv, tp0, tp1, to0, to1, tbl, sem, isem0, isem1, osem0, osem1):
        sid = lax.axis_index("s")
        cc = lax.axis_index("c")
        wid = sid * 2 + cc

        def in_src(blk):
            return g0_hbm.at[:, pl.ds(sid * pos_per_tile + blk * _K, _K)]

        def out_dst(blk):
            return tbl.at[cc, pl.ds(sid * pos_per_tile + blk * _K, _K)]

        pltpu.async_copy(in_src(0), tp0, isem0)
        pltpu.async_copy(in_src(1), tp1, isem1)

        def tpair(b2, _):
            for par, (tpb, tob, isem, osem) in enumerate(
                    ((tp0, to0, isem0, osem0), (tp1, to1, isem1, osem1))):
                blk = b2 * 2 + par
                pltpu.make_async_copy(in_src(blk), tpb, isem).wait()

                @pl.when(blk >= 2)
                def _():
                    pltpu.make_async_copy(tob, out_dst(blk - 2), osem).wait()

                def shuffle(gg, _):
                    lane = lax.iota(jnp.int32, _L)
                    srows = lane & 7
                    ssel = lane >> 3
                    for u in range(8):
                        g = gg * 8 + u
                        v = plsc.load_gather(tpb, [srows, ssel + 2 * g])
                        plsc.store_scatter(tob, [ssel + 2 * g, srows], v)
                    return 0

                lax.fori_loop(0, (_K * 8 // _L) // 8, shuffle, 0)
                pltpu.async_copy(tob, out_dst(blk), osem)

                @pl.when(blk + 2 < blocks)
                def _():
                    pltpu.async_copy(in_src(blk + 2), tpb, isem)
            return 0

        lax.fori_loop(0, blocks // 2, tpair, 0)
        pltpu.make_async_copy(to0, out_dst(blocks - 2), osem0).wait()
        pltpu.make_async_copy(to1, out_dst(blocks - 1), osem1).wait()
        plsc.subcore_barrier()

        def step(st, _):
            base = wid * per_w + st * _B
            pltpu.sync_copy(ux_hbm.at[pl.ds(base, _B)], uxv)
            pltpu.sync_copy(uy_hbm.at[pl.ds(base, _B)], uyv)

            for g in range(_B // _L):
                sl = pl.ds(g * _L, _L)
                x = uxv[sl]
                y = uyv[sl]
                fx = _floor_i32(x * res0 - 0.5)
                fy = _floor_i32(y * res0 - 0.5)
                x0 = jnp.clip(fx, 0, res0 - 2)
                y0 = jnp.clip(fy, 0, res0 - 2)
                b00 = y0 * res0 + x0
                idx[0, sl] = b00
                idx[1, sl] = b00 + 1
                idx[2, sl] = b00 + res0
                idx[3, sl] = b00 + res0 + 1
                qx = x * res1 - 0.5
                qy = y * res1 - 0.5
                ix0 = _floor_i32(qx)
                iy0 = _floor_i32(qy)
                wx1 = qx - ix0.astype(jnp.float32)
                wy1 = qy - iy0.astype(jnp.float32)
                wx0 = 1.0 - wx1
                wy0 = 1.0 - wy1
                wx0 = jnp.where(ix0 >= 0, wx0, 0.0)
                wy0 = jnp.where(iy0 >= 0, wy0, 0.0)
                wx1 = jnp.where(ix0 + 1 <= res1 - 1, wx1, 0.0)
                wy1 = jnp.where(iy0 + 1 <= res1 - 1, wy1, 0.0)
                jx0 = jnp.maximum(ix0, 0)
                jy0 = jnp.maximum(iy0, 0)
                jx1 = jnp.minimum(ix0 + 1, res1 - 1)
                jy1 = jnp.minimum(iy0 + 1, res1 - 1)
                idx[4, sl] = jy0 * res1 + jx0
                idx[5, sl] = jy0 * res1 + jx1
                idx[6, sl] = jy1 * res1 + jx0
                idx[7, sl] = jy1 * res1 + jx1
                wts[0, sl] = wy0 * wx0
                wts[1, sl] = wy0 * wx1
                wts[2, sl] = wy1 * wx0
                wts[3, sl] = wy1 * wx1

            cps = [
                pltpu.async_copy(tbl.at[cc].at[idx.at[0]], c00, sem),
                pltpu.async_copy(tbl.at[cc].at[idx.at[1]], c01, sem),
                pltpu.async_copy(tbl.at[cc].at[idx.at[2]], c10, sem),
                pltpu.async_copy(tbl.at[cc].at[idx.at[3]], c11, sem),
                pltpu.async_copy(t1_hbm.at[idx.at[4]], s00, sem),
                pltpu.async_copy(t1_hbm.at[idx.at[5]], s01, sem),
                pltpu.async_copy(t1_hbm.at[idx.at[6]], s10, sem),
                pltpu.async_copy(t1_hbm.at[idx.at[7]], s11, sem),
            ]
            for cp in cps:
                cp.wait()

            def group(g, _):
                g16 = g * _L
                lane = lax.iota(jnp.int32, _L)
                rowsel = lane >> 3
                colsrc = lane & 7
                for p in range(_L // 2):
                    rows = rowsel + (g16 + 2 * p)
                    for kk, cbuf in enumerate((c00, c01, c10, c11)):
                        v = plsc.load_gather(cbuf, [rows, colsrc])
                        plsc.store_scatter(outv, [rows, colsrc + 8 * kk], v)
                wv = [wts[kk, pl.ds(g16, _L)] for kk in range(4)]
                for t in range(_L):
                    i = g16 + t
                    acc = (s00[i, :] * wv[0][t] + s01[i, :] * wv[1][t]
                           + s10[i, :] * wv[2][t] + s11[i, :] * wv[3][t])
                    outv[i, 32:48] = acc
                return 0

            lax.fori_loop(0, _B // _L, group, 0)
            pltpu.sync_copy(outv, out_hbm.at[pl.ds(base, _B)])
            return 0

        lax.fori_loop(0, steps, step, 0)

    return k(ux, uy, g0f, t1)


def kernel(uv, g0, g1):
    c0, res0 = g0.shape[1], g0.shape[2]
    c1, res1 = g1.shape[1], g1.shape[2]
    n = uv.shape[0]
    g0f = g0.reshape(c0, res0 * res0)
    t1 = jnp.transpose(g1[0], (1, 2, 0)).reshape(res1 * res1, c1)
    ux = uv[:, 0] + 0.0
    uy = uv[:, 1] + 0.0
    return _feature_level_sc(ux, uy, g0f, t1, n, res0, res1, c0, c1)
